# baseline (speedup 1.0000x reference)
import functools

import jax
import jax.numpy as jnp
from jax import lax
from jax.experimental import pallas as pl
from jax.experimental.pallas import tpu as pltpu
from jax.experimental.pallas import tpu_sc as plsc

_B, _S, _D = 16, 2048, 64
_V = 100000
_N = _B * _S
_NW = 32
_RPW = _N // _NW
_CHUNK = 512
_NCHUNK = _RPW // _CHUNK
_G = 128
_NG = _CHUNK // _G
_L = 16

_NSLAB = _V // _G
_VT = _NSLAB * _G
_SPT = -(-_NSLAB // _NW)

_mesh = plsc.VectorSubcoreMesh(core_axis_name="c", subcore_axis_name="s")
_params = pltpu.CompilerParams(
    use_tc_tiling_on_sc=True, needs_layout_passes=False
)


@functools.partial(
    pl.kernel,
    mesh=_mesh,
    out_type=jax.ShapeDtypeStruct((_V, 128), jnp.float32),
    scratch_types=[
        pltpu.VMEM((_D, _G), jnp.float32),
        pltpu.VMEM((_G, _G), jnp.float32),
        pltpu.VMEM((32 // 2, 128), jnp.float32),
        pltpu.VMEM((32, 128), jnp.float32),
    ],
    compiler_params=_params,
)
def _format_table(tokt_hbm, tail_hbm, out_hbm, src_v, dst_v, tsrc_v, tdst_v):
    cid = lax.axis_index("c")
    sid = lax.axis_index("s")
    wid = sid * 2 + cid

    lanes = lax.iota(jnp.int32, _L)

    def _slab(j, carry):
        sl = wid + _NW * j

        @pl.when(sl < _NSLAB)
        def _():
            v0 = pl.multiple_of(sl * _G, _G)
            pltpu.sync_copy(tokt_hbm.at[:, pl.ds(v0, _G)], src_v)
            for c in range(_G // _L):
                jb = c * _L + lanes
                for f in range(_D):
                    fv = jnp.full((_L,), f, jnp.int32)
                    plsc.store_scatter(
                        dst_v, [jb, fv], src_v[f, pl.ds(c * _L, _L)]
                    )
            pltpu.sync_copy(dst_v, out_hbm.at[pl.ds(v0, _G)])

        return carry

    lax.fori_loop(0, _SPT, _slab, 0)

    @pl.when(wid == 0)
    def _tail():
        pltpu.sync_copy(tail_hbm, tsrc_v)
        for v in range(_V - _VT):
            for c in range(_D // _L):
                tdst_v[v, pl.ds(c * _L, _L)] = tsrc_v[
                    v // 2, pl.ds((v % 2) * _D + c * _L, _L)
                ]
        pltpu.sync_copy(tdst_v, out_hbm.at[pl.ds(_VT, _V - _VT)])


@functools.partial(
    pl.kernel,
    mesh=_mesh,
    out_type=jax.ShapeDtypeStruct((_N // 2, 128), jnp.float32),
    scratch_types=[
        pltpu.VMEM((_RPW // _G, _G), jnp.int32),
        pltpu.VMEM((_CHUNK, 128), jnp.float32),
        pltpu.VMEM((_CHUNK // 2, 128), jnp.float32),
        pltpu.SemaphoreType.DMA,
    ],
    compiler_params=_params,
)
def _emb_lookup(x_hbm, tok_hbm, pos_hbm, out_hbm, ids_v, gat_v, pos_v, sem):
    cid = lax.axis_index("c")
    sid = lax.axis_index("s")
    wid = sid * 2 + cid
    base = wid * _RPW
    pos_base = lax.rem(base, _S)

    nrow = _RPW // _G
    x0 = pl.multiple_of(wid * nrow, 8)
    pltpu.sync_copy(x_hbm.at[pl.ds(x0, nrow)], ids_v)

    for k in range(_NCHUNK):
        cps = [
            pltpu.async_copy(
                tok_hbm.at[ids_v.at[k * _NG + g]],
                gat_v.at[pl.ds(g * _G, _G)],
                sem,
            )
            for g in range(_NG)
        ]
        p0 = pl.multiple_of((pos_base + k * _CHUNK) // 2, 8)
        pltpu.sync_copy(pos_hbm.at[pl.ds(p0, _CHUNK // 2)], pos_v)
        for cp in cps:
            cp.wait()

        def _add_row(r2, carry):
            for h in range(2):
                for c in range(_D // _L):
                    dst = pl.ds(h * _D + c * _L, _L)
                    src = pl.ds(c * _L, _L)
                    pos_v[r2, dst] = pos_v[r2, dst] + gat_v[2 * r2 + h, src]
            return carry

        lax.fori_loop(0, _CHUNK // 2, _add_row, 0)

        out0 = pl.multiple_of((base + k * _CHUNK) // 2, 8)
        pltpu.sync_copy(pos_v, out_hbm.at[pl.ds(out0, _CHUNK // 2)])


def kernel(x, token_emb, pos_emb):
    idx = x.astype(jnp.int32).reshape(_N // _G, _G)
    tok_t = token_emb.T
    tail = token_emb[_VT:, :].reshape((_V - _VT) // 2, 128)
    tok128 = _format_table(tok_t, tail)
    pos2 = pos_emb.reshape(_S // 2, 128)
    out = _emb_lookup(idx, tok128, pos2)
    return out.reshape(_B, _S, _D)


# speedup vs baseline: 1.4769x; 1.4769x over previous
import functools

import jax
import jax.numpy as jnp
from jax import lax
from jax.experimental import pallas as pl
from jax.experimental.pallas import tpu as pltpu
from jax.experimental.pallas import tpu_sc as plsc

_B, _S, _D = 16, 2048, 64
_V = 100000
_N = _B * _S
_NW = 32
_RPW = _N // _NW
_CHUNK = 512
_NCHUNK = _RPW // _CHUNK
_G = 128
_NG = _CHUNK // _G
_L = 16

_SLAB = 256
_VPT = _V // _NW
_NSL = -(-_VPT // _SLAB)

_mesh = plsc.VectorSubcoreMesh(core_axis_name="c", subcore_axis_name="s")
_params = pltpu.CompilerParams(
    use_tc_tiling_on_sc=True, needs_layout_passes=False
)


@functools.partial(
    pl.kernel,
    mesh=_mesh,
    out_type=jax.ShapeDtypeStruct((_V, 128), jnp.float32),
    scratch_types=[
        pltpu.VMEM((_SLAB, _D), jnp.float32),
        pltpu.VMEM((_SLAB, 128), jnp.float32),
    ],
    compiler_params=_params,
)
def _widen_table(tok_hbm, out_hbm, src_v, dst_v):
    cid = lax.axis_index("c")
    sid = lax.axis_index("s")
    wid = sid * 2 + cid
    lo = pl.multiple_of((wid * _VPT + 7) // 8 * 8, 8)
    hi = ((wid + 1) * _VPT + 7) // 8 * 8
    hi = jnp.where(wid == _NW - 1, _V, hi)

    def _slab(j, carry):
        v0 = lo + j * _SLAB

        @pl.when(v0 < hi)
        def _():
            vc = pl.multiple_of(jnp.minimum(v0, _V - _SLAB), 8)
            pltpu.sync_copy(tok_hbm.at[pl.ds(vc, _SLAB)], src_v)

            def _row(r, c2):
                for c in range(_D // _L):
                    sl = pl.ds(c * _L, _L)
                    dst_v[r, sl] = src_v[r, sl]
                return c2

            lax.fori_loop(0, _SLAB, _row, 0)
            pltpu.sync_copy(dst_v, out_hbm.at[pl.ds(vc, _SLAB)])

        return carry

    lax.fori_loop(0, _NSL, _slab, 0)


@functools.partial(
    pl.kernel,
    mesh=_mesh,
    out_type=jax.ShapeDtypeStruct((_N // 2, 128), jnp.float32),
    scratch_types=[
        pltpu.VMEM((_RPW // _G, _G), jnp.int32),
        pltpu.VMEM((_CHUNK, 128), jnp.float32),
        pltpu.VMEM((_CHUNK // 2, 128), jnp.float32),
        pltpu.SemaphoreType.DMA,
    ],
    compiler_params=_params,
)
def _emb_lookup(x_hbm, tok_hbm, pos_hbm, out_hbm, ids_v, gat_v, pos_v, sem):
    cid = lax.axis_index("c")
    sid = lax.axis_index("s")
    wid = sid * 2 + cid
    base = wid * _RPW
    pos_base = lax.rem(base, _S)

    nrow = _RPW // _G
    x0 = pl.multiple_of(wid * nrow, 8)
    pltpu.sync_copy(x_hbm.at[pl.ds(x0, nrow)], ids_v)

    for k in range(_NCHUNK):
        cps = [
            pltpu.async_copy(
                tok_hbm.at[ids_v.at[k * _NG + g]],
                gat_v.at[pl.ds(g * _G, _G)],
                sem,
            )
            for g in range(_NG)
        ]
        p0 = pl.multiple_of((pos_base + k * _CHUNK) // 2, 8)
        pltpu.sync_copy(pos_hbm.at[pl.ds(p0, _CHUNK // 2)], pos_v)
        for cp in cps:
            cp.wait()

        def _add_row(r2, carry):
            for h in range(2):
                for c in range(_D // _L):
                    dst = pl.ds(h * _D + c * _L, _L)
                    src = pl.ds(c * _L, _L)
                    pos_v[r2, dst] = pos_v[r2, dst] + gat_v[2 * r2 + h, src]
            return carry

        lax.fori_loop(0, _CHUNK // 2, _add_row, 0)

        out0 = pl.multiple_of((base + k * _CHUNK) // 2, 8)
        pltpu.sync_copy(pos_v, out_hbm.at[pl.ds(out0, _CHUNK // 2)])


def kernel(x, token_emb, pos_emb):
    idx = x.astype(jnp.int32).reshape(_N // _G, _G)
    tok128 = _widen_table(token_emb)
    pos2 = pos_emb.reshape(_S // 2, 128)
    out = _emb_lookup(idx, tok128, pos2)
    return out.reshape(_B, _S, _D)


# speedup vs baseline: 2.0468x; 1.3858x over previous
import functools

import jax
import jax.numpy as jnp
from jax import lax
from jax.experimental import pallas as pl
from jax.experimental.pallas import tpu as pltpu
from jax.experimental.pallas import tpu_sc as plsc

_B, _S, _D = 16, 2048, 64
_N = _B * _S
_NW = 32
_RPW = _N // _NW
_G = 128
_NG = _RPW // _G
_CHUNK = 256
_NCHUNK = _RPW // _CHUNK
_L = 16

_mesh = plsc.VectorSubcoreMesh(core_axis_name="c", subcore_axis_name="s")


@functools.partial(
    pl.kernel,
    mesh=_mesh,
    out_type=jax.ShapeDtypeStruct((_N // 2, 128), jnp.float32),
    scratch_types=[
        pltpu.VMEM((_NG, _G), jnp.int32),
        pltpu.VMEM((_RPW, _D), jnp.float32),
        pltpu.VMEM((_CHUNK // 2, 128), jnp.float32),
        pltpu.VMEM((_CHUNK // 2, 128), jnp.float32),
        pltpu.SemaphoreType.DMA,
        pltpu.SemaphoreType.DMA,
    ],
    compiler_params=pltpu.CompilerParams(use_tc_tiling_on_sc=False),
)
def _emb_lookup(x_hbm, tok_hbm, pos_hbm, out_hbm, ids_v, rows_v, pb0, pb1,
                gsem, ssem):
    cid = lax.axis_index("c")
    sid = lax.axis_index("s")
    wid = sid * 2 + cid
    base = wid * _RPW
    pos_base = lax.rem(base, _S)

    x0 = pl.multiple_of(wid * _NG, 8)
    pltpu.sync_copy(x_hbm.at[pl.ds(x0, _NG)], ids_v)

    gcps = [
        pltpu.async_copy(
            tok_hbm.at[ids_v.at[g]],
            rows_v.at[pl.ds(g * _G, _G)],
            gsem,
        )
        for g in range(_NG)
    ]

    pbs = [pb0, pb1]
    scps = [None] * _NCHUNK
    gpc = _CHUNK // _G
    for k in range(_NCHUNK):
        pb = pbs[k % 2]
        if k >= 2:
            scps[k - 2].wait()
        p0 = pl.multiple_of((pos_base + k * _CHUNK) // 2, 8)
        pltpu.sync_copy(pos_hbm.at[pl.ds(p0, _CHUNK // 2)], pb)
        for g in range(gpc):
            gcps[k * gpc + g].wait()

        @plsc.parallel_loop(0, _CHUNK // 2, unroll=4)
        def _add(r2):
            for h in range(2):
                for c in range(_D // _L):
                    dst = pl.ds(h * _D + c * _L, _L)
                    src = pl.ds(c * _L, _L)
                    pb[r2, dst] = pb[r2, dst] + rows_v[
                        k * _CHUNK + 2 * r2 + h, src
                    ]

        out0 = pl.multiple_of((base + k * _CHUNK) // 2, 8)
        scps[k] = pltpu.async_copy(pb, out_hbm.at[pl.ds(out0, _CHUNK // 2)],
                                   ssem)
    scps[_NCHUNK - 2].wait()
    scps[_NCHUNK - 1].wait()


def kernel(x, token_emb, pos_emb):
    idx = x.astype(jnp.int32).reshape(_N // _G, _G)
    pos2 = pos_emb.reshape(_S // 2, 128)
    out = _emb_lookup(idx, token_emb, pos2)
    return out.reshape(_B, _S, _D)
